# baseline (speedup 1.0000x reference)
import functools

import jax
import jax.numpy as jnp
from jax import lax
from jax.experimental import pallas as pl
from jax.experimental.pallas import tpu as pltpu
from jax.experimental.pallas import tpu_sc as plsc

N = 10000
E = 160000
F_IN = 128
HEADS = 8
HIDDEN = 64
CLASSES = 40
ORDER = 16

NSC = 2
NSUB = 16
NPAD = 10240
RPS = NPAD // NSUB
TS = 64
NT = RPS // TS
B = 128
EPS = E // NSUB
NB = 80
JUNK = N
NBD = 160

_f32 = jnp.float32
_mesh = plsc.VectorSubcoreMesh(core_axis_name="c", subcore_axis_name="s")


def _fill_zero(zbuf, nv):
    zero = jnp.zeros((16,), _f32)

    @pl.loop(0, TS)
    def _(i):
        for v in range(nv):
            zbuf[i, pl.ds(16 * v, 16)] = zero


def _cheb_body(nch, chunk, kind, *refs):
    nv = chunk // 16
    rounds = nch // NSC
    if kind == "first_deg":
        (h_hbm, tha_hbm, thb_hbm, src_hbm, dst_hbm, alli_hbm,
         g0_hbm, gn_hbm, acc_hbm, d16_hbm, dsq16_hbm, sd16_hbm,
         src_v, dst_v, rows_v, rows_w, zbuf, sv, gv, av, dsq_v, aux_v,
         th_a, th_b, s_sp, sem_g0, sem_g1, sem_s0, sem_s1,
         alli_v, st16) = refs
    elif kind == "first":
        (h_hbm, d_hbm, dsq_hbm, tha_hbm, thb_hbm, src_hbm, dst_hbm,
         g0_hbm, gn_hbm, acc_hbm,
         src_v, dst_v, rows_v, rows_w, zbuf, sv, gv, av, dsq_v, aux_v,
         th_a, th_b, s_sp, sem_g0, sem_g1, sem_s0, sem_s1) = refs
    elif kind == "mid":
        (gcur_hbm, gprev_hbm, accin_hbm, dsq_hbm, tha_hbm, src_hbm, dst_hbm,
         gn_hbm, acc_hbm,
         src_v, dst_v, rows_v, rows_w, zbuf, sv, gv, av, dsq_v,
         th_a, s_sp, sem_g0, sem_g1, sem_s0, sem_s1) = refs
    else:
        (gcur_hbm, gprev_hbm, accin_hbm, dsq_hbm, sd_hbm, tha_hbm, src_hbm,
         dst_hbm,
         acc_hbm,
         src_v, dst_v, rows_v, rows_w, zbuf, sv, gv, av, dsq_v, aux_v,
         th_a, s_sp, sem_g0, sem_g1, sem_s0, sem_s1) = refs

    c = lax.axis_index("c")
    s = lax.axis_index("s")
    rbase = s * RPS

    lane16 = pl.ds(0, 16)
    if kind != "first_deg":
        pltpu.sync_copy(dsq_hbm.at[pl.ds(rbase, RPS)], dsq_v)
    if kind == "first":
        pltpu.sync_copy(d_hbm.at[pl.ds(rbase, RPS)], aux_v)
    elif kind == "final":
        pltpu.sync_copy(sd_hbm.at[pl.ds(rbase, RPS)], aux_v)
    pltpu.sync_copy(dst_hbm.at[s], dst_v)
    pltpu.sync_copy(tha_hbm, th_a)
    if kind in ("first", "first_deg"):
        pltpu.sync_copy(thb_hbm, th_b)
    _fill_zero(zbuf, nv)

    if kind == "first_deg":
        one = jnp.ones((16,), _f32)

        @pl.loop(0, B)
        def _(i):
            for v in range(nv):
                rows_v[i, pl.ds(16 * v, 16)] = one

        for t in range(NT):
            pltpu.sync_copy(zbuf, s_sp.at[pl.ds(rbase + t * TS, TS)])
        plsc.subcore_barrier()

        for kk in range(NBD // 32):
            pltpu.sync_copy(alli_hbm.at[s, pl.ds(kk * 32, 32)], alli_v)

            @pl.loop(0, 32)
            def _(j):
                pltpu.async_copy(rows_v, s_sp.at[alli_v.at[j]], sem_s0,
                                 add=True)

            @pl.loop(0, 32)
            def _(j):
                pltpu.make_async_copy(
                    rows_v, s_sp.at[alli_v.at[j]], sem_s0).wait()

        plsc.subcore_barrier()
        magic = jnp.full((16,), 0x5F3759DF, jnp.int32)
        half = jnp.full((16,), 0.5, _f32)
        threehalf = jnp.full((16,), 1.5, _f32)
        for t in range(NT):
            r0 = rbase + t * TS
            pltpu.sync_copy(s_sp.at[pl.ds(r0, TS)], sv)

            @pl.loop(0, TS)
            def _(i, t=t):
                x = jnp.maximum(sv[i, lane16], 1.0)
                y = plsc.bitcast(
                    magic - lax.shift_right_arithmetic(
                        plsc.bitcast(x, jnp.int32), 1), _f32)
                hx = half * x
                for _it in range(3):
                    y = y * (threehalf - hx * y * y)
                dsq_v[t * TS + i, lane16] = y * y
                aux_v[t * TS + i, lane16] = y
                st16[i, lane16] = x * y

            @pl.when(c == 0)
            def _(t=t, r0=r0):
                pltpu.sync_copy(st16, sd16_hbm.at[pl.ds(r0, TS)])

        @pl.when(c == 0)
        def _():
            pltpu.sync_copy(aux_v, d16_hbm.at[pl.ds(rbase, RPS)])
            pltpu.sync_copy(dsq_v, dsq16_hbm.at[pl.ds(rbase, RPS)])

    coef = -1.0 if kind in ("first", "first_deg") else -2.0
    for r in range(rounds):
        ch = c * rounds + r
        gb = ch * NPAD
        pltpu.sync_copy(src_hbm.at[ch, s], src_v)
        tva = [th_a[ch, pl.ds(16 * v, 16)] for v in range(nv)]
        first = kind in ("first", "first_deg")
        if first:
            tvb = [th_b[ch, pl.ds(16 * v, 16)] for v in range(nv)]

        if first:
            for t in range(NT):
                r0 = rbase + t * TS
                pltpu.sync_copy(h_hbm.at[pl.ds(gb + r0, TS)], gv)

                @pl.loop(0, TS)
                def _(i, t=t):
                    dd = aux_v[t * TS + i, lane16]
                    for v in range(nv):
                        cs = pl.ds(16 * v, 16)
                        gv[i, cs] = dd * gv[i, cs]

                pltpu.sync_copy(gv, g0_hbm.at[pl.ds(gb + r0, TS)])

        for t in range(NT):
            pltpu.sync_copy(zbuf, s_sp.at[pl.ds(rbase + t * TS, TS)])
        plsc.subcore_barrier()

        gsrc = g0_hbm if first else gcur_hbm

        def _gather(j, buf, sem):
            return pltpu.async_copy(gsrc.at[src_v.at[j]], buf, sem)

        def _scatter(j, buf, sem):
            return pltpu.async_copy(buf, s_sp.at[dst_v.at[j]], sem, add=True)

        _gather(0, rows_v, sem_g0)

        @pl.loop(0, NB // 2)
        def _(jj):
            j0 = 2 * jj
            j1 = j0 + 1

            @pl.when(jj > 0)
            def _():
                pltpu.make_async_copy(
                    rows_w, s_sp.at[dst_v.at[j0 - 1]], sem_s1).wait()

            _gather(j1, rows_w, sem_g1)
            pltpu.make_async_copy(gsrc.at[src_v.at[j0]], rows_v, sem_g0).wait()
            _scatter(j0, rows_v, sem_s0)
            pltpu.make_async_copy(rows_v, s_sp.at[dst_v.at[j0]], sem_s0).wait()

            @pl.when(jj < NB // 2 - 1)
            def _():
                _gather(j0 + 2, rows_v, sem_g0)

            pltpu.make_async_copy(gsrc.at[src_v.at[j1]], rows_w, sem_g1).wait()
            _scatter(j1, rows_w, sem_s1)

        pltpu.make_async_copy(
            rows_w, s_sp.at[dst_v.at[NB - 1]], sem_s1).wait()
        plsc.subcore_barrier()

        for t in range(NT):
            r0 = rbase + t * TS
            g0r = gb + r0
            pltpu.sync_copy(s_sp.at[pl.ds(r0, TS)], sv)
            gp_src = g0_hbm if first else gprev_hbm
            pltpu.sync_copy(gp_src.at[pl.ds(g0r, TS)], gv)
            if not first:
                pltpu.sync_copy(accin_hbm.at[pl.ds(g0r, TS)], av)

            @pl.loop(0, TS)
            def _(i, t=t):
                m = coef * dsq_v[t * TS + i, lane16]
                if kind == "final":
                    sdd = aux_v[t * TS + i, lane16]
                for v in range(nv):
                    cs = pl.ds(16 * v, 16)
                    gn = m * sv[i, cs]
                    if first:
                        a = tva[v] * gv[i, cs] + tvb[v] * gn
                    else:
                        gn = gn - gv[i, cs]
                        a = av[i, cs] + tva[v] * gn
                    if kind == "final":
                        a = sdd * a
                    av[i, cs] = a
                    if kind != "final":
                        gv[i, cs] = gn

            if kind != "final":
                pltpu.sync_copy(gv, gn_hbm.at[pl.ds(g0r, TS)])
            pltpu.sync_copy(av, acc_hbm.at[pl.ds(g0r, TS)])
        plsc.subcore_barrier()


def _make_cheb_step(nch, chunk, kind):
    R = nch * NPAD
    n_out = {"first_deg": 3, "first": 3, "mid": 2, "final": 1}[kind]
    outs = tuple(jax.ShapeDtypeStruct((R, chunk), _f32) for _ in range(n_out))
    if kind == "first_deg":
        outs = outs + tuple(
            jax.ShapeDtypeStruct((NPAD, 16), _f32) for _ in range(3))
    scratch = (
        pltpu.VMEM((NB, B), jnp.int32),
        pltpu.VMEM((NB, B), jnp.int32),
        pltpu.VMEM((B, chunk), _f32),
        pltpu.VMEM((B, chunk), _f32),
        pltpu.VMEM((TS, chunk), _f32),
        pltpu.VMEM((TS, chunk), _f32),
        pltpu.VMEM((TS, chunk), _f32),
        pltpu.VMEM((TS, chunk), _f32),
        pltpu.VMEM((RPS, 16), _f32),
    )
    if kind != "mid":
        scratch = scratch + (pltpu.VMEM((RPS, 16), _f32),)
    scratch = scratch + (pltpu.VMEM((nch, chunk), _f32),)
    if kind in ("first", "first_deg"):
        scratch = scratch + (pltpu.VMEM((nch, chunk), _f32),)
    scratch = scratch + (
        pltpu.VMEM_SHARED((NPAD, chunk), _f32),
        pltpu.SemaphoreType.DMA,
        pltpu.SemaphoreType.DMA,
        pltpu.SemaphoreType.DMA,
        pltpu.SemaphoreType.DMA,
    )
    if kind == "first_deg":
        scratch = scratch + (
            pltpu.VMEM((32, B), jnp.int32),
            pltpu.VMEM((TS, 16), _f32),
        )
    return pl.kernel(
        functools.partial(_cheb_body, nch, chunk, kind),
        out_type=outs,
        mesh=_mesh,
        scratch_types=scratch,
        compiler_params=pltpu.CompilerParams(
            use_tc_tiling_on_sc=False,
            needs_layout_passes=False,
        ),
    )


def _mm1_body(x_ref, w_ref, o_ref):
    o_ref[...] = jnp.dot(x_ref[...], w_ref[...],
                         preferred_element_type=_f32)


def _mm1(x_pad, W1):
    bm = 1024
    nb = NPAD // bm
    return pl.pallas_call(
        _mm1_body,
        out_shape=jax.ShapeDtypeStruct((4 * NPAD, 128), _f32),
        grid=(nb, 4),
        in_specs=[
            pl.BlockSpec((bm, F_IN), lambda i, ch: (i, 0)),
            pl.BlockSpec((F_IN, 128), lambda i, ch: (0, ch)),
        ],
        out_specs=pl.BlockSpec((bm, 128), lambda i, ch, nb=nb: (ch * nb + i, 0)),
    )(x_pad, W1)


def _mid_body(acc_ref, b_ref, w_ref, l1_ref, h2_ref):
    a = acc_ref[...] + b_ref[0:1, :]
    l1 = jnp.where(a > 0, a, jnp.exp(a) - 1.0)
    l1_ref[...] = l1

    @pl.when(pl.program_id(1) == 0)
    def _():
        h2_ref[...] = jnp.zeros_like(h2_ref)

    h2_ref[...] += jnp.dot(l1, w_ref[...], preferred_element_type=_f32)


def _mid(accT1, b1_2d, W2pad):
    bm = 1024
    nb = NPAD // bm
    return pl.pallas_call(
        _mid_body,
        out_shape=(
            jax.ShapeDtypeStruct((NPAD, HEADS * HIDDEN), _f32),
            jax.ShapeDtypeStruct((NPAD, 128), _f32),
        ),
        grid=(nb, 4),
        in_specs=[
            pl.BlockSpec((bm, 128), lambda i, ch, nb=nb: (ch * nb + i, 0)),
            pl.BlockSpec((8, 128), lambda i, ch: (ch, 0)),
            pl.BlockSpec((128, 128), lambda i, ch: (ch, 0)),
        ],
        out_specs=(
            pl.BlockSpec((bm, 128), lambda i, ch: (i, ch)),
            pl.BlockSpec((bm, 128), lambda i, ch: (i, 0)),
        ),
    )(accT1, b1_2d, W2pad)


def _final_body(x_ref, o_ref):
    x = x_ref[...]
    e = jnp.where(x > 0, x, jnp.exp(x) - 1.0)
    col = lax.broadcasted_iota(jnp.int32, x.shape, 1)
    valid = col < CLASSES
    em = jnp.where(valid, e, -1e30)
    m = jnp.max(em, axis=1, keepdims=True)
    z = jnp.where(valid, jnp.exp(em - m), 0.0)
    lse = jnp.log(jnp.sum(z, axis=1, keepdims=True))
    o_ref[...] = em - m - lse


def _final(l2pad):
    bm = 1024
    return pl.pallas_call(
        _final_body,
        out_shape=jax.ShapeDtypeStruct((NPAD, 128), _f32),
        grid=(NPAD // bm,),
        in_specs=[pl.BlockSpec((bm, 128), lambda i: (i, 0))],
        out_specs=pl.BlockSpec((bm, 128), lambda i: (i, 0)),
    )(l2pad)


_step1_l1 = _make_cheb_step(8, 64, "first_deg")
_step_l1 = _make_cheb_step(8, 64, "mid")
_stepF_l1 = _make_cheb_step(8, 64, "final")
_step1_l2 = _make_cheb_step(2, 32, "first")
_step_l2 = _make_cheb_step(2, 32, "mid")
_stepF_l2 = _make_cheb_step(2, 32, "final")


def _cheb_rest(g0, g1, acc, dsq16, sd16, th, src_off, dstp,
               mid_fn, final_fn):
    gprev, gcur = g0, g1
    for k in range(2, ORDER):
        gn, acc = mid_fn(gcur, gprev, acc, dsq16, th[k], src_off, dstp)
        gprev, gcur = gcur, gn
    (accT,) = final_fn(gcur, gprev, acc, dsq16, sd16, th[ORDER],
                       src_off, dstp)
    return accT


def kernel(x, edge_index, W1, b1, theta1, W2, b2, theta2):
    src = edge_index[0]
    dst = edge_index[1]
    padw = NB * B - EPS

    srcp = jnp.pad(src.reshape(NSUB, EPS), ((0, 0), (0, padw)),
                   constant_values=JUNK).reshape(NSUB, NB, B)
    dstp = jnp.pad(dst.reshape(NSUB, EPS), ((0, 0), (0, padw)),
                   constant_values=JUNK).reshape(NSUB, NB, B)
    src1 = srcp[None] + (jnp.arange(8, dtype=jnp.int32) * NPAD)[:, None, None, None]
    src2 = srcp[None] + (jnp.arange(2, dtype=jnp.int32) * NPAD)[:, None, None, None]
    padd = NBD * B - 2 * EPS
    alli = jnp.pad(jnp.concatenate([src, dst]).reshape(NSUB, 2 * EPS),
                   ((0, 0), (0, padd)),
                   constant_values=JUNK).reshape(NSUB, NBD, B)

    th1 = jnp.repeat(theta1, HIDDEN, axis=0).T.reshape(ORDER + 1, 8, 64)
    th2 = jnp.broadcast_to(theta2.T, (ORDER + 1, 64)).reshape(ORDER + 1, 2, 32)

    x_pad = jnp.pad(x, ((0, NPAD - N), (0, 0)))
    h1c128 = _mm1(x_pad, W1)
    h1c = (h1c128.reshape(4, NPAD, 2, 64).transpose(0, 2, 1, 3)
           .reshape(8 * NPAD, 64))
    g0, g1, acc, d16, dsq16, sd16 = _step1_l1(h1c, th1[0], th1[1],
                                              src1, dstp, alli)
    accT1 = _cheb_rest(g0, g1, acc, dsq16, sd16, th1, src1, dstp,
                       _step_l1, _stepF_l1)
    accT1c = (accT1.reshape(4, 2, NPAD, 64).transpose(0, 2, 1, 3)
              .reshape(4 * NPAD, 128))

    b1_2d = jnp.broadcast_to(b1.reshape(4, 1, 128), (4, 8, 128)).reshape(32, 128)
    W2pad = jnp.pad(W2.reshape(HEADS * HIDDEN, CLASSES),
                    ((0, 0), (0, 128 - CLASSES)))
    layer1_pad, h2 = _mid(accT1c, b1_2d, W2pad)
    layer1 = layer1_pad[:N]

    h2c = h2[:, :64].reshape(NPAD, 2, 32).transpose(1, 0, 2).reshape(2 * NPAD, 32)
    g0b, g1b, acc2 = _step1_l2(h2c, d16, dsq16, th2[0], th2[1], src2, dstp)
    accT2 = _cheb_rest(g0b, g1b, acc2, dsq16, sd16, th2, src2, dstp,
                       _step_l2, _stepF_l2)
    accT2_std = accT2.reshape(2, NPAD, 32).transpose(1, 0, 2).reshape(NPAD, 64)
    layer2 = accT2_std[:N, :CLASSES] + b2

    l2pad = jnp.pad(accT2_std, ((0, 0), (0, 64)))
    l2pad = l2pad + jnp.pad(b2, (0, 88))[None, :]
    logp = _final(l2pad)[:N, :CLASSES]
    return (logp, layer2, layer1)


# speedup vs baseline: 1.0571x; 1.0571x over previous
import functools

import jax
import jax.numpy as jnp
from jax import lax
from jax.experimental import pallas as pl
from jax.experimental.pallas import tpu as pltpu
from jax.experimental.pallas import tpu_sc as plsc

N = 10000
E = 160000
F_IN = 128
HEADS = 8
HIDDEN = 64
CLASSES = 40
ORDER = 16

NSC = 2
NSUB = 16
NPAD = 10240
RPS = NPAD // NSUB
ZH = 64
NZ = RPS // ZH
B = 128
EPS = E // NSUB
NB = 80
JUNK = N
NBD = 160

_f32 = jnp.float32
_mesh = plsc.VectorSubcoreMesh(core_axis_name="c", subcore_axis_name="s")


def _fill_zero(zbuf, nv):
    zero = jnp.zeros((16,), _f32)

    @pl.loop(0, ZH)
    def _(i):
        for v in range(nv):
            zbuf[i, pl.ds(16 * v, 16)] = zero


def _cheb_body(nch, chunk, kind, *refs):
    nv = chunk // 16
    rounds = nch // NSC
    TS = 64 if kind in ("first", "first_deg") else 128
    NT = RPS // TS
    if kind == "first_deg":
        (h_hbm, tha_hbm, thb_hbm, src_hbm, dst_hbm, alli_hbm,
         g0_hbm, gn_hbm, acc_hbm, d16_hbm, dsq16_hbm, sd16_hbm,
         src_v, dst_v, rows_v, rows_w, zbuf, sv, gv, av, dsq_v, aux_v,
         th_a, th_b, s_sp, sem_g0, sem_g1, sem_s0, sem_s1,
         alli_v, st16) = refs
    elif kind == "first":
        (h_hbm, d_hbm, dsq_hbm, tha_hbm, thb_hbm, src_hbm, dst_hbm,
         g0_hbm, gn_hbm, acc_hbm,
         src_v, dst_v, rows_v, rows_w, zbuf, sv, gv, av, dsq_v, aux_v,
         th_a, th_b, s_sp, sem_g0, sem_g1, sem_s0, sem_s1) = refs
    elif kind == "mid":
        (gcur_hbm, gprev_hbm, accin_hbm, dsq_hbm, tha_hbm, src_hbm, dst_hbm,
         gn_hbm, acc_hbm,
         src_v, dst_v, rows_v, rows_w, zbuf, sv, gv, av, dsq_v,
         th_a, s_sp, sem_g0, sem_g1, sem_s0, sem_s1) = refs
    else:
        (gcur_hbm, gprev_hbm, accin_hbm, dsq_hbm, sd_hbm, tha_hbm, src_hbm,
         dst_hbm,
         acc_hbm,
         src_v, dst_v, rows_v, rows_w, zbuf, sv, gv, av, dsq_v, aux_v,
         th_a, s_sp, sem_g0, sem_g1, sem_s0, sem_s1) = refs

    c = lax.axis_index("c")
    s = lax.axis_index("s")
    rbase = s * RPS

    lane16 = pl.ds(0, 16)
    if kind != "first_deg":
        pltpu.sync_copy(dsq_hbm.at[pl.ds(rbase, RPS)], dsq_v)
    if kind == "first":
        pltpu.sync_copy(d_hbm.at[pl.ds(rbase, RPS)], aux_v)
    elif kind == "final":
        pltpu.sync_copy(sd_hbm.at[pl.ds(rbase, RPS)], aux_v)
    pltpu.sync_copy(dst_hbm.at[s], dst_v)
    pltpu.sync_copy(tha_hbm, th_a)
    if kind in ("first", "first_deg"):
        pltpu.sync_copy(thb_hbm, th_b)
    _fill_zero(zbuf, nv)

    if kind == "first_deg":
        one = jnp.ones((16,), _f32)

        @pl.loop(0, B)
        def _(i):
            for v in range(nv):
                rows_v[i, pl.ds(16 * v, 16)] = one

        for t in range(NZ):
            pltpu.sync_copy(zbuf, s_sp.at[pl.ds(rbase + t * ZH, ZH)])
        plsc.subcore_barrier()

        for kk in range(NBD // 32):
            pltpu.sync_copy(alli_hbm.at[s, pl.ds(kk * 32, 32)], alli_v)

            @pl.loop(0, 32)
            def _(j):
                pltpu.async_copy(rows_v, s_sp.at[alli_v.at[j]], sem_s0,
                                 add=True)

            @pl.loop(0, 32)
            def _(j):
                pltpu.make_async_copy(
                    rows_v, s_sp.at[alli_v.at[j]], sem_s0).wait()

        plsc.subcore_barrier()
        magic = jnp.full((16,), 0x5F3759DF, jnp.int32)
        half = jnp.full((16,), 0.5, _f32)
        threehalf = jnp.full((16,), 1.5, _f32)
        for t in range(NT):
            r0 = rbase + t * TS
            pltpu.sync_copy(s_sp.at[pl.ds(r0, TS)], sv)

            @pl.loop(0, TS)
            def _(i, t=t):
                x = jnp.maximum(sv[i, lane16], 1.0)
                y = plsc.bitcast(
                    magic - lax.shift_right_arithmetic(
                        plsc.bitcast(x, jnp.int32), 1), _f32)
                hx = half * x
                for _it in range(3):
                    y = y * (threehalf - hx * y * y)
                dsq_v[t * TS + i, lane16] = y * y
                aux_v[t * TS + i, lane16] = y
                st16[i, lane16] = x * y

            @pl.when(c == 0)
            def _(t=t, r0=r0):
                pltpu.sync_copy(st16, sd16_hbm.at[pl.ds(r0, TS)])

        @pl.when(c == 0)
        def _():
            pltpu.sync_copy(aux_v, d16_hbm.at[pl.ds(rbase, RPS)])
            pltpu.sync_copy(dsq_v, dsq16_hbm.at[pl.ds(rbase, RPS)])

    coef = -1.0 if kind in ("first", "first_deg") else -2.0
    for r in range(rounds):
        ch = c * rounds + r
        gb = ch * NPAD
        pltpu.sync_copy(src_hbm.at[ch, s], src_v)
        tva = [th_a[ch, pl.ds(16 * v, 16)] for v in range(nv)]
        first = kind in ("first", "first_deg")
        if first:
            tvb = [th_b[ch, pl.ds(16 * v, 16)] for v in range(nv)]

        if first:
            for t in range(NT):
                r0 = rbase + t * TS
                pltpu.sync_copy(h_hbm.at[pl.ds(gb + r0, TS)], gv)

                @pl.loop(0, TS)
                def _(i, t=t):
                    dd = aux_v[t * TS + i, lane16]
                    for v in range(nv):
                        cs = pl.ds(16 * v, 16)
                        gv[i, cs] = dd * gv[i, cs]

                pltpu.sync_copy(gv, g0_hbm.at[pl.ds(gb + r0, TS)])

        for t in range(NZ):
            pltpu.sync_copy(zbuf, s_sp.at[pl.ds(rbase + t * ZH, ZH)])
        plsc.subcore_barrier()

        gsrc = g0_hbm if first else gcur_hbm

        def _gather(j, buf, sem):
            return pltpu.async_copy(gsrc.at[src_v.at[j]], buf, sem)

        def _scatter(j, buf, sem):
            return pltpu.async_copy(buf, s_sp.at[dst_v.at[j]], sem, add=True)

        _gather(0, rows_v, sem_g0)

        @pl.loop(0, NB // 2)
        def _(jj):
            j0 = 2 * jj
            j1 = j0 + 1

            @pl.when(jj > 0)
            def _():
                pltpu.make_async_copy(
                    rows_w, s_sp.at[dst_v.at[j0 - 1]], sem_s1).wait()

            _gather(j1, rows_w, sem_g1)
            pltpu.make_async_copy(gsrc.at[src_v.at[j0]], rows_v, sem_g0).wait()
            _scatter(j0, rows_v, sem_s0)
            pltpu.make_async_copy(rows_v, s_sp.at[dst_v.at[j0]], sem_s0).wait()

            @pl.when(jj < NB // 2 - 1)
            def _():
                _gather(j0 + 2, rows_v, sem_g0)

            pltpu.make_async_copy(gsrc.at[src_v.at[j1]], rows_w, sem_g1).wait()
            _scatter(j1, rows_w, sem_s1)

        pltpu.make_async_copy(
            rows_w, s_sp.at[dst_v.at[NB - 1]], sem_s1).wait()
        plsc.subcore_barrier()

        for t in range(NT):
            r0 = rbase + t * TS
            g0r = gb + r0
            pltpu.sync_copy(s_sp.at[pl.ds(r0, TS)], sv)
            gp_src = g0_hbm if first else gprev_hbm
            pltpu.sync_copy(gp_src.at[pl.ds(g0r, TS)], gv)
            if not first:
                pltpu.sync_copy(accin_hbm.at[pl.ds(g0r, TS)], av)

            @pl.loop(0, TS)
            def _(i, t=t):
                m = coef * dsq_v[t * TS + i, lane16]
                if kind == "final":
                    sdd = aux_v[t * TS + i, lane16]
                for v in range(nv):
                    cs = pl.ds(16 * v, 16)
                    gn = m * sv[i, cs]
                    if first:
                        a = tva[v] * gv[i, cs] + tvb[v] * gn
                    else:
                        gn = gn - gv[i, cs]
                        a = av[i, cs] + tva[v] * gn
                    if kind == "final":
                        a = sdd * a
                    av[i, cs] = a
                    if kind != "final":
                        gv[i, cs] = gn

            if kind != "final":
                pltpu.sync_copy(gv, gn_hbm.at[pl.ds(g0r, TS)])
            pltpu.sync_copy(av, acc_hbm.at[pl.ds(g0r, TS)])
        plsc.subcore_barrier()


def _make_cheb_step(nch, chunk, kind):
    ts = 64 if kind in ("first", "first_deg") else 128
    R = nch * NPAD
    n_out = {"first_deg": 3, "first": 3, "mid": 2, "final": 1}[kind]
    outs = tuple(jax.ShapeDtypeStruct((R, chunk), _f32) for _ in range(n_out))
    if kind == "first_deg":
        outs = outs + tuple(
            jax.ShapeDtypeStruct((NPAD, 16), _f32) for _ in range(3))
    scratch = (
        pltpu.VMEM((NB, B), jnp.int32),
        pltpu.VMEM((NB, B), jnp.int32),
        pltpu.VMEM((B, chunk), _f32),
        pltpu.VMEM((B, chunk), _f32),
        pltpu.VMEM((ZH, chunk), _f32),
        pltpu.VMEM((ts, chunk), _f32),
        pltpu.VMEM((ts, chunk), _f32),
        pltpu.VMEM((ts, chunk), _f32),
        pltpu.VMEM((RPS, 16), _f32),
    )
    if kind != "mid":
        scratch = scratch + (pltpu.VMEM((RPS, 16), _f32),)
    scratch = scratch + (pltpu.VMEM((nch, chunk), _f32),)
    if kind in ("first", "first_deg"):
        scratch = scratch + (pltpu.VMEM((nch, chunk), _f32),)
    scratch = scratch + (
        pltpu.VMEM_SHARED((NPAD, chunk), _f32),
        pltpu.SemaphoreType.DMA,
        pltpu.SemaphoreType.DMA,
        pltpu.SemaphoreType.DMA,
        pltpu.SemaphoreType.DMA,
    )
    if kind == "first_deg":
        scratch = scratch + (
            pltpu.VMEM((32, B), jnp.int32),
            pltpu.VMEM((64, 16), _f32),
        )
    return pl.kernel(
        functools.partial(_cheb_body, nch, chunk, kind),
        out_type=outs,
        mesh=_mesh,
        scratch_types=scratch,
        compiler_params=pltpu.CompilerParams(
            use_tc_tiling_on_sc=False,
            needs_layout_passes=False,
        ),
    )


def _mm1_body(x_ref, w_ref, o_ref):
    o_ref[...] = jnp.dot(x_ref[...], w_ref[...],
                         preferred_element_type=_f32)


def _mm1(x_pad, W1):
    bm = 1024
    nb = NPAD // bm
    return pl.pallas_call(
        _mm1_body,
        out_shape=jax.ShapeDtypeStruct((4 * NPAD, 128), _f32),
        grid=(nb, 4),
        in_specs=[
            pl.BlockSpec((bm, F_IN), lambda i, ch: (i, 0)),
            pl.BlockSpec((F_IN, 128), lambda i, ch: (0, ch)),
        ],
        out_specs=pl.BlockSpec((bm, 128), lambda i, ch, nb=nb: (ch * nb + i, 0)),
    )(x_pad, W1)


def _mid_body(acc_ref, b_ref, w_ref, l1_ref, h2_ref):
    a = acc_ref[...] + b_ref[0:1, :]
    l1 = jnp.where(a > 0, a, jnp.exp(a) - 1.0)
    l1_ref[...] = l1

    @pl.when(pl.program_id(1) == 0)
    def _():
        h2_ref[...] = jnp.zeros_like(h2_ref)

    h2_ref[...] += jnp.dot(l1, w_ref[...], preferred_element_type=_f32)


def _mid(accT1, b1_2d, W2pad):
    bm = 1024
    nb = NPAD // bm
    return pl.pallas_call(
        _mid_body,
        out_shape=(
            jax.ShapeDtypeStruct((NPAD, HEADS * HIDDEN), _f32),
            jax.ShapeDtypeStruct((NPAD, 128), _f32),
        ),
        grid=(nb, 4),
        in_specs=[
            pl.BlockSpec((bm, 128), lambda i, ch, nb=nb: (ch * nb + i, 0)),
            pl.BlockSpec((8, 128), lambda i, ch: (ch, 0)),
            pl.BlockSpec((128, 128), lambda i, ch: (ch, 0)),
        ],
        out_specs=(
            pl.BlockSpec((bm, 128), lambda i, ch: (i, ch)),
            pl.BlockSpec((bm, 128), lambda i, ch: (i, 0)),
        ),
    )(accT1, b1_2d, W2pad)


def _final_body(x_ref, o_ref):
    x = x_ref[...]
    e = jnp.where(x > 0, x, jnp.exp(x) - 1.0)
    col = lax.broadcasted_iota(jnp.int32, x.shape, 1)
    valid = col < CLASSES
    em = jnp.where(valid, e, -1e30)
    m = jnp.max(em, axis=1, keepdims=True)
    z = jnp.where(valid, jnp.exp(em - m), 0.0)
    lse = jnp.log(jnp.sum(z, axis=1, keepdims=True))
    o_ref[...] = em - m - lse


def _final(l2pad):
    bm = 1024
    return pl.pallas_call(
        _final_body,
        out_shape=jax.ShapeDtypeStruct((NPAD, 128), _f32),
        grid=(NPAD // bm,),
        in_specs=[pl.BlockSpec((bm, 128), lambda i: (i, 0))],
        out_specs=pl.BlockSpec((bm, 128), lambda i: (i, 0)),
    )(l2pad)


_step1_l1 = _make_cheb_step(8, 64, "first_deg")
_step_l1 = _make_cheb_step(8, 64, "mid")
_stepF_l1 = _make_cheb_step(8, 64, "final")
_step1_l2 = _make_cheb_step(2, 32, "first")
_step_l2 = _make_cheb_step(2, 32, "mid")
_stepF_l2 = _make_cheb_step(2, 32, "final")


def _cheb_rest(g0, g1, acc, dsq16, sd16, th, src_off, dstp,
               mid_fn, final_fn):
    gprev, gcur = g0, g1
    for k in range(2, ORDER):
        gn, acc = mid_fn(gcur, gprev, acc, dsq16, th[k], src_off, dstp)
        gprev, gcur = gcur, gn
    (accT,) = final_fn(gcur, gprev, acc, dsq16, sd16, th[ORDER],
                       src_off, dstp)
    return accT


def kernel(x, edge_index, W1, b1, theta1, W2, b2, theta2):
    src = edge_index[0]
    dst = edge_index[1]
    padw = NB * B - EPS

    srcp = jnp.pad(src.reshape(NSUB, EPS), ((0, 0), (0, padw)),
                   constant_values=JUNK).reshape(NSUB, NB, B)
    dstp = jnp.pad(dst.reshape(NSUB, EPS), ((0, 0), (0, padw)),
                   constant_values=JUNK).reshape(NSUB, NB, B)
    src1 = srcp[None] + (jnp.arange(8, dtype=jnp.int32) * NPAD)[:, None, None, None]
    src2 = srcp[None] + (jnp.arange(2, dtype=jnp.int32) * NPAD)[:, None, None, None]
    padd = NBD * B - 2 * EPS
    alli = jnp.pad(jnp.concatenate([src, dst]).reshape(NSUB, 2 * EPS),
                   ((0, 0), (0, padd)),
                   constant_values=JUNK).reshape(NSUB, NBD, B)

    th1 = jnp.repeat(theta1, HIDDEN, axis=0).T.reshape(ORDER + 1, 8, 64)
    th2 = jnp.broadcast_to(theta2.T, (ORDER + 1, 64)).reshape(ORDER + 1, 2, 32)

    x_pad = jnp.pad(x, ((0, NPAD - N), (0, 0)))
    h1c128 = _mm1(x_pad, W1)
    h1c = (h1c128.reshape(4, NPAD, 2, 64).transpose(0, 2, 1, 3)
           .reshape(8 * NPAD, 64))
    g0, g1, acc, d16, dsq16, sd16 = _step1_l1(h1c, th1[0], th1[1],
                                              src1, dstp, alli)
    accT1 = _cheb_rest(g0, g1, acc, dsq16, sd16, th1, src1, dstp,
                       _step_l1, _stepF_l1)
    accT1c = (accT1.reshape(4, 2, NPAD, 64).transpose(0, 2, 1, 3)
              .reshape(4 * NPAD, 128))

    b1_2d = jnp.broadcast_to(b1.reshape(4, 1, 128), (4, 8, 128)).reshape(32, 128)
    W2pad = jnp.pad(W2.reshape(HEADS * HIDDEN, CLASSES),
                    ((0, 0), (0, 128 - CLASSES)))
    layer1_pad, h2 = _mid(accT1c, b1_2d, W2pad)
    layer1 = layer1_pad[:N]

    h2c = h2[:, :64].reshape(NPAD, 2, 32).transpose(1, 0, 2).reshape(2 * NPAD, 32)
    g0b, g1b, acc2 = _step1_l2(h2c, d16, dsq16, th2[0], th2[1], src2, dstp)
    accT2 = _cheb_rest(g0b, g1b, acc2, dsq16, sd16, th2, src2, dstp,
                       _step_l2, _stepF_l2)
    accT2_std = accT2.reshape(2, NPAD, 32).transpose(1, 0, 2).reshape(NPAD, 64)
    layer2 = accT2_std[:N, :CLASSES] + b2

    l2pad = jnp.pad(accT2_std, ((0, 0), (0, 64)))
    l2pad = l2pad + jnp.pad(b2, (0, 88))[None, :]
    logp = _final(l2pad)[:N, :CLASSES]
    return (logp, layer2, layer1)


# speedup vs baseline: 1.2329x; 1.1664x over previous
import functools

import jax
import jax.numpy as jnp
from jax import lax
from jax.experimental import pallas as pl
from jax.experimental.pallas import tpu as pltpu
from jax.experimental.pallas import tpu_sc as plsc

N = 10000
E = 160000
F_IN = 128
HEADS = 8
HIDDEN = 64
CLASSES = 40
ORDER = 16

NSC = 2
NSUB = 16
NPAD = 10240
RPS = NPAD // NSUB
ZH = 64
NZ = RPS // ZH
B = 128
EPS = E // NSUB
NB = 80
JUNK = N
NBD = 160

_f32 = jnp.float32
_mesh = plsc.VectorSubcoreMesh(core_axis_name="c", subcore_axis_name="s")


def _fill_zero(zbuf, nv):
    zero = jnp.zeros((16,), _f32)

    @pl.loop(0, ZH)
    def _(i):
        for v in range(nv):
            zbuf[i, pl.ds(16 * v, 16)] = zero


def _cheb_body(nch, chunk, kind, *refs):
    nv = chunk // 16
    rounds = nch // NSC
    TS = 128 if kind == "mid" else 64
    NT = RPS // TS
    if kind == "first_deg":
        (h_hbm, tha_hbm, thb_hbm, src_hbm, dst_hbm, alli_hbm,
         g0_hbm, gn_hbm, acc_hbm, d16_hbm, dsq16_hbm, sd16_hbm,
         src_v, dst_v, rows_v, rows_w, rows_x, rows_y, sv, gv, av,
         dsq_v, aux_v,
         th_a, th_b, s_sp, sem_g0, sem_g1, sem_s0, sem_s1,
         alli_v, st16) = refs
    elif kind == "first":
        (h_hbm, d_hbm, dsq_hbm, tha_hbm, thb_hbm, src_hbm, dst_hbm,
         g0_hbm, gn_hbm, acc_hbm,
         src_v, dst_v, rows_v, rows_w, rows_x, rows_y, sv, gv, av,
         dsq_v, aux_v,
         th_a, th_b, s_sp, sem_g0, sem_g1, sem_s0, sem_s1) = refs
    elif kind == "mid":
        (gcur_hbm, gprev_hbm, accin_hbm, dsq_hbm, tha_hbm, src_hbm, dst_hbm,
         gn_hbm, acc_hbm,
         src_v, dst_v, rows_v, rows_w, rows_x, rows_y, sv, gv, av,
         dsq_v,
         th_a, s_sp, sem_g0, sem_g1, sem_s0, sem_s1) = refs
    else:
        (gcur_hbm, gprev_hbm, accin_hbm, dsq_hbm, sd_hbm, tha_hbm, src_hbm,
         dst_hbm,
         acc_hbm,
         src_v, dst_v, rows_v, rows_w, rows_x, rows_y, sv, gv, av,
         dsq_v, aux_v,
         th_a, s_sp, sem_g0, sem_g1, sem_s0, sem_s1) = refs

    c = lax.axis_index("c")
    s = lax.axis_index("s")
    rbase = s * RPS

    lane16 = pl.ds(0, 16)
    if kind != "first_deg":
        pltpu.sync_copy(dsq_hbm.at[pl.ds(rbase, RPS)], dsq_v)
    if kind == "first":
        pltpu.sync_copy(d_hbm.at[pl.ds(rbase, RPS)], aux_v)
    elif kind == "final":
        pltpu.sync_copy(sd_hbm.at[pl.ds(rbase, RPS)], aux_v)
    pltpu.sync_copy(dst_hbm.at[s], dst_v)
    pltpu.sync_copy(tha_hbm, th_a)
    if kind in ("first", "first_deg"):
        pltpu.sync_copy(thb_hbm, th_b)
    _fill_zero(av, nv)
    zslice = av.at[pl.ds(0, ZH)]

    if kind == "first_deg":
        one = jnp.ones((16,), _f32)

        @pl.loop(0, B)
        def _(i):
            for v in range(nv):
                rows_v[i, pl.ds(16 * v, 16)] = one

        for t in range(NZ):
            pltpu.sync_copy(zslice, s_sp.at[pl.ds(rbase + t * ZH, ZH)])
        plsc.subcore_barrier()

        for kk in range(NBD // 16):
            pltpu.sync_copy(alli_hbm.at[s, pl.ds(kk * 16, 16)], alli_v)

            @pl.loop(0, 16)
            def _(j):
                pltpu.async_copy(rows_v, s_sp.at[alli_v.at[j]], sem_s0,
                                 add=True)

            @pl.loop(0, 16)
            def _(j):
                pltpu.make_async_copy(
                    rows_v, s_sp.at[alli_v.at[j]], sem_s0).wait()

        plsc.subcore_barrier()
        magic = jnp.full((16,), 0x5F3759DF, jnp.int32)
        half = jnp.full((16,), 0.5, _f32)
        threehalf = jnp.full((16,), 1.5, _f32)
        for t in range(NT):
            r0 = rbase + t * TS
            pltpu.sync_copy(s_sp.at[pl.ds(r0, TS)], sv)

            @pl.loop(0, TS)
            def _(i, t=t):
                x = jnp.maximum(sv[i, lane16], 1.0)
                y = plsc.bitcast(
                    magic - lax.shift_right_arithmetic(
                        plsc.bitcast(x, jnp.int32), 1), _f32)
                hx = half * x
                for _it in range(3):
                    y = y * (threehalf - hx * y * y)
                dsq_v[t * TS + i, lane16] = y * y
                aux_v[t * TS + i, lane16] = y
                st16[i, lane16] = x * y

            @pl.when(c == 0)
            def _(t=t, r0=r0):
                pltpu.sync_copy(st16, sd16_hbm.at[pl.ds(r0, TS)])

        @pl.when(c == 0)
        def _():
            pltpu.sync_copy(aux_v, d16_hbm.at[pl.ds(rbase, RPS)])
            pltpu.sync_copy(dsq_v, dsq16_hbm.at[pl.ds(rbase, RPS)])

    coef = -1.0 if kind in ("first", "first_deg") else -2.0
    for r in range(rounds):
        ch = c * rounds + r
        gb = ch * NPAD
        pltpu.sync_copy(src_hbm.at[ch, s], src_v)
        tva = [th_a[ch, pl.ds(16 * v, 16)] for v in range(nv)]
        first = kind in ("first", "first_deg")
        if first:
            tvb = [th_b[ch, pl.ds(16 * v, 16)] for v in range(nv)]

        if first:
            for t in range(NT):
                r0 = rbase + t * TS
                pltpu.sync_copy(h_hbm.at[pl.ds(gb + r0, TS)], gv)

                @pl.loop(0, TS)
                def _(i, t=t):
                    dd = aux_v[t * TS + i, lane16]
                    for v in range(nv):
                        cs = pl.ds(16 * v, 16)
                        gv[i, cs] = dd * gv[i, cs]

                pltpu.sync_copy(gv, g0_hbm.at[pl.ds(gb + r0, TS)])

        if r > 0:
            _fill_zero(av, nv)
        for t in range(NZ):
            pltpu.sync_copy(zslice, s_sp.at[pl.ds(rbase + t * ZH, ZH)])
        plsc.subcore_barrier()

        gsrc = g0_hbm if first else gcur_hbm

        def _gather(j, buf, sem):
            return pltpu.async_copy(gsrc.at[src_v.at[j]], buf, sem)

        def _scatter(j, buf, sem):
            return pltpu.async_copy(buf, s_sp.at[dst_v.at[j]], sem, add=True)

        bufs = (rows_v, rows_w, rows_x, rows_y)
        sems = (sem_g0, sem_g1, sem_s0, sem_s1)

        def _wait_g(j, u):
            pltpu.make_async_copy(gsrc.at[src_v.at[j]], bufs[u],
                                  sems[u]).wait()

        def _wait_s(j, u):
            pltpu.make_async_copy(bufs[u], s_sp.at[dst_v.at[j]],
                                  sems[u]).wait()

        _gather(0, bufs[0], sems[0])
        _gather(1, bufs[1], sems[1])

        @pl.loop(0, NB // 4)
        def _(jj):
            j = 4 * jj
            for u in range(4):
                ju = j + u

                @pl.when(ju >= 2)
                def _(ju=ju, u=u):
                    _wait_s(ju - 2, (u + 2) % 4)

                @pl.when(ju + 2 < NB)
                def _(ju=ju, u=u):
                    _gather(ju + 2, bufs[(u + 2) % 4], sems[(u + 2) % 4])

                _wait_g(ju, u)
                _scatter(ju, bufs[u], sems[u])

        _wait_s(NB - 2, (NB - 2) % 4)
        _wait_s(NB - 1, (NB - 1) % 4)
        plsc.subcore_barrier()

        for t in range(NT):
            r0 = rbase + t * TS
            g0r = gb + r0
            pltpu.sync_copy(s_sp.at[pl.ds(r0, TS)], sv)
            gp_src = g0_hbm if first else gprev_hbm
            pltpu.sync_copy(gp_src.at[pl.ds(g0r, TS)], gv)
            if not first:
                pltpu.sync_copy(accin_hbm.at[pl.ds(g0r, TS)], av)

            @pl.loop(0, TS)
            def _(i, t=t):
                m = coef * dsq_v[t * TS + i, lane16]
                if kind == "final":
                    sdd = aux_v[t * TS + i, lane16]
                for v in range(nv):
                    cs = pl.ds(16 * v, 16)
                    gn = m * sv[i, cs]
                    if first:
                        a = tva[v] * gv[i, cs] + tvb[v] * gn
                    else:
                        gn = gn - gv[i, cs]
                        a = av[i, cs] + tva[v] * gn
                    if kind == "final":
                        a = sdd * a
                    av[i, cs] = a
                    if kind != "final":
                        gv[i, cs] = gn

            if kind != "final":
                pltpu.sync_copy(gv, gn_hbm.at[pl.ds(g0r, TS)])
            pltpu.sync_copy(av, acc_hbm.at[pl.ds(g0r, TS)])
        plsc.subcore_barrier()


def _make_cheb_step(nch, chunk, kind):
    ts = 128 if kind == "mid" else 64
    R = nch * NPAD
    n_out = {"first_deg": 3, "first": 3, "mid": 2, "final": 1}[kind]
    outs = tuple(jax.ShapeDtypeStruct((R, chunk), _f32) for _ in range(n_out))
    if kind == "first_deg":
        outs = outs + tuple(
            jax.ShapeDtypeStruct((NPAD, 16), _f32) for _ in range(3))
    scratch = (
        pltpu.VMEM((NB, B), jnp.int32),
        pltpu.VMEM((NB, B), jnp.int32),
        pltpu.VMEM((B, chunk), _f32),
        pltpu.VMEM((B, chunk), _f32),
        pltpu.VMEM((B, chunk), _f32),
        pltpu.VMEM((B, chunk), _f32),
        pltpu.VMEM((ts, chunk), _f32),
        pltpu.VMEM((ts, chunk), _f32),
        pltpu.VMEM((ts, chunk), _f32),
        pltpu.VMEM((RPS, 16), _f32),
    )
    if kind != "mid":
        scratch = scratch + (pltpu.VMEM((RPS, 16), _f32),)
    scratch = scratch + (pltpu.VMEM((nch, chunk), _f32),)
    if kind in ("first", "first_deg"):
        scratch = scratch + (pltpu.VMEM((nch, chunk), _f32),)
    scratch = scratch + (
        pltpu.VMEM_SHARED((NPAD, chunk), _f32),
        pltpu.SemaphoreType.DMA,
        pltpu.SemaphoreType.DMA,
        pltpu.SemaphoreType.DMA,
        pltpu.SemaphoreType.DMA,
    )
    if kind == "first_deg":
        scratch = scratch + (
            pltpu.VMEM((16, B), jnp.int32),
            pltpu.VMEM((64, 16), _f32),
        )
    return pl.kernel(
        functools.partial(_cheb_body, nch, chunk, kind),
        out_type=outs,
        mesh=_mesh,
        scratch_types=scratch,
        compiler_params=pltpu.CompilerParams(
            use_tc_tiling_on_sc=False,
            needs_layout_passes=False,
        ),
    )


def _mm1_body(x_ref, w_ref, o_ref):
    o_ref[...] = jnp.dot(x_ref[...], w_ref[...],
                         preferred_element_type=_f32)


def _mm1(x_pad, W1):
    bm = 1024
    nb = NPAD // bm
    return pl.pallas_call(
        _mm1_body,
        out_shape=jax.ShapeDtypeStruct((4 * NPAD, 128), _f32),
        grid=(nb, 4),
        in_specs=[
            pl.BlockSpec((bm, F_IN), lambda i, ch: (i, 0)),
            pl.BlockSpec((F_IN, 128), lambda i, ch: (0, ch)),
        ],
        out_specs=pl.BlockSpec((bm, 128), lambda i, ch, nb=nb: (ch * nb + i, 0)),
    )(x_pad, W1)


def _mid_body(acc_ref, b_ref, w_ref, l1_ref, h2_ref):
    a = acc_ref[...] + b_ref[0:1, :]
    l1 = jnp.where(a > 0, a, jnp.exp(a) - 1.0)
    l1_ref[...] = l1

    @pl.when(pl.program_id(1) == 0)
    def _():
        h2_ref[...] = jnp.zeros_like(h2_ref)

    h2_ref[...] += jnp.dot(l1, w_ref[...], preferred_element_type=_f32)


def _mid(accT1, b1_2d, W2pad):
    bm = 1024
    nb = NPAD // bm
    return pl.pallas_call(
        _mid_body,
        out_shape=(
            jax.ShapeDtypeStruct((NPAD, HEADS * HIDDEN), _f32),
            jax.ShapeDtypeStruct((NPAD, 128), _f32),
        ),
        grid=(nb, 4),
        in_specs=[
            pl.BlockSpec((bm, 128), lambda i, ch, nb=nb: (ch * nb + i, 0)),
            pl.BlockSpec((8, 128), lambda i, ch: (ch, 0)),
            pl.BlockSpec((128, 128), lambda i, ch: (ch, 0)),
        ],
        out_specs=(
            pl.BlockSpec((bm, 128), lambda i, ch: (i, ch)),
            pl.BlockSpec((bm, 128), lambda i, ch: (i, 0)),
        ),
    )(accT1, b1_2d, W2pad)


def _final_body(x_ref, o_ref):
    x = x_ref[...]
    e = jnp.where(x > 0, x, jnp.exp(x) - 1.0)
    col = lax.broadcasted_iota(jnp.int32, x.shape, 1)
    valid = col < CLASSES
    em = jnp.where(valid, e, -1e30)
    m = jnp.max(em, axis=1, keepdims=True)
    z = jnp.where(valid, jnp.exp(em - m), 0.0)
    lse = jnp.log(jnp.sum(z, axis=1, keepdims=True))
    o_ref[...] = em - m - lse


def _final(l2pad):
    bm = 1024
    return pl.pallas_call(
        _final_body,
        out_shape=jax.ShapeDtypeStruct((NPAD, 128), _f32),
        grid=(NPAD // bm,),
        in_specs=[pl.BlockSpec((bm, 128), lambda i: (i, 0))],
        out_specs=pl.BlockSpec((bm, 128), lambda i: (i, 0)),
    )(l2pad)


_step1_l1 = _make_cheb_step(8, 64, "first_deg")
_step_l1 = _make_cheb_step(8, 64, "mid")
_stepF_l1 = _make_cheb_step(8, 64, "final")
_step1_l2 = _make_cheb_step(2, 32, "first")
_step_l2 = _make_cheb_step(2, 32, "mid")
_stepF_l2 = _make_cheb_step(2, 32, "final")


def _cheb_rest(g0, g1, acc, dsq16, sd16, th, src_off, dstp,
               mid_fn, final_fn):
    gprev, gcur = g0, g1
    for k in range(2, ORDER):
        gn, acc = mid_fn(gcur, gprev, acc, dsq16, th[k], src_off, dstp)
        gprev, gcur = gcur, gn
    (accT,) = final_fn(gcur, gprev, acc, dsq16, sd16, th[ORDER],
                       src_off, dstp)
    return accT


def kernel(x, edge_index, W1, b1, theta1, W2, b2, theta2):
    src = edge_index[0]
    dst = edge_index[1]
    padw = NB * B - EPS

    srcp = jnp.pad(src.reshape(NSUB, EPS), ((0, 0), (0, padw)),
                   constant_values=JUNK).reshape(NSUB, NB, B)
    dstp = jnp.pad(dst.reshape(NSUB, EPS), ((0, 0), (0, padw)),
                   constant_values=JUNK).reshape(NSUB, NB, B)
    src1 = srcp[None] + (jnp.arange(8, dtype=jnp.int32) * NPAD)[:, None, None, None]
    src2 = srcp[None] + (jnp.arange(2, dtype=jnp.int32) * NPAD)[:, None, None, None]
    padd = NBD * B - 2 * EPS
    alli = jnp.pad(jnp.concatenate([src, dst]).reshape(NSUB, 2 * EPS),
                   ((0, 0), (0, padd)),
                   constant_values=JUNK).reshape(NSUB, NBD, B)

    th1 = jnp.repeat(theta1, HIDDEN, axis=0).T.reshape(ORDER + 1, 8, 64)
    th2 = jnp.broadcast_to(theta2.T, (ORDER + 1, 64)).reshape(ORDER + 1, 2, 32)

    x_pad = jnp.pad(x, ((0, NPAD - N), (0, 0)))
    h1c128 = _mm1(x_pad, W1)
    h1c = (h1c128.reshape(4, NPAD, 2, 64).transpose(0, 2, 1, 3)
           .reshape(8 * NPAD, 64))
    g0, g1, acc, d16, dsq16, sd16 = _step1_l1(h1c, th1[0], th1[1],
                                              src1, dstp, alli)
    accT1 = _cheb_rest(g0, g1, acc, dsq16, sd16, th1, src1, dstp,
                       _step_l1, _stepF_l1)
    accT1c = (accT1.reshape(4, 2, NPAD, 64).transpose(0, 2, 1, 3)
              .reshape(4 * NPAD, 128))

    b1_2d = jnp.broadcast_to(b1.reshape(4, 1, 128), (4, 8, 128)).reshape(32, 128)
    W2pad = jnp.pad(W2.reshape(HEADS * HIDDEN, CLASSES),
                    ((0, 0), (0, 128 - CLASSES)))
    layer1_pad, h2 = _mid(accT1c, b1_2d, W2pad)
    layer1 = layer1_pad[:N]

    h2c = h2[:, :64].reshape(NPAD, 2, 32).transpose(1, 0, 2).reshape(2 * NPAD, 32)
    g0b, g1b, acc2 = _step1_l2(h2c, d16, dsq16, th2[0], th2[1], src2, dstp)
    accT2 = _cheb_rest(g0b, g1b, acc2, dsq16, sd16, th2, src2, dstp,
                       _step_l2, _stepF_l2)
    accT2_std = accT2.reshape(2, NPAD, 32).transpose(1, 0, 2).reshape(NPAD, 64)
    layer2 = accT2_std[:N, :CLASSES] + b2

    l2pad = jnp.pad(accT2_std, ((0, 0), (0, 64)))
    l2pad = l2pad + jnp.pad(b2, (0, 88))[None, :]
    logp = _final(l2pad)[:N, :CLASSES]
    return (logp, layer2, layer1)


# speedup vs baseline: 1.2618x; 1.0234x over previous
import functools

import jax
import jax.numpy as jnp
from jax import lax
from jax.experimental import pallas as pl
from jax.experimental.pallas import tpu as pltpu
from jax.experimental.pallas import tpu_sc as plsc

N = 10000
E = 160000
F_IN = 128
HEADS = 8
HIDDEN = 64
CLASSES = 40
ORDER = 16

NSC = 2
NSUB = 16
NPAD = 10240
RPS = NPAD // NSUB
ZH = 64
NZ = RPS // ZH
B = 128
EPS = E // NSUB
NB = 80
JUNK = N
NBD = 160

_f32 = jnp.float32
_mesh = plsc.VectorSubcoreMesh(core_axis_name="c", subcore_axis_name="s")


def _fill_zero(zbuf, nv):
    zero = jnp.zeros((16,), _f32)

    @pl.loop(0, ZH)
    def _(i):
        for v in range(nv):
            zbuf[i, pl.ds(16 * v, 16)] = zero


def _cheb_body(nch, chunk, kind, *refs):
    nv = chunk // 16
    rounds = nch // NSC
    TS = 128 if kind in ("mid", "pair") else 64
    NT = RPS // TS
    if kind == "first_deg":
        (h_hbm, tha_hbm, thb_hbm, src_hbm, dst_hbm, alli_hbm,
         g0_hbm, gn_hbm, acc_hbm, d16_hbm, dsq16_hbm, sd16_hbm,
         src_v, dst_v, rows_v, rows_w, rows_x, rows_y, sv, gv, av,
         dsq_v, aux_v,
         th_a, th_b, s_sp, sem_g0, sem_g1, sem_s0, sem_s1,
         alli_v, st16) = refs
    elif kind == "first":
        (h_hbm, d_hbm, dsq_hbm, tha_hbm, thb_hbm, src_hbm, dst_hbm,
         g0_hbm, gn_hbm, acc_hbm,
         src_v, dst_v, rows_v, rows_w, rows_x, rows_y, sv, gv, av,
         dsq_v, aux_v,
         th_a, th_b, s_sp, sem_g0, sem_g1, sem_s0, sem_s1) = refs
    elif kind == "pair":
        (gcur_hbm, gprev_hbm, accin_hbm, dsq_hbm, tha_hbm, thb_hbm,
         src_hbm, dst_hbm,
         ga_hbm, gb_hbm, acc_hbm,
         src_v, dst_v, rows_v, rows_w, rows_x, rows_y, sv, gv, av,
         dsq_v,
         th_a, th_b, s_sp, sem_g0, sem_g1, sem_s0, sem_s1) = refs
    elif kind == "mid":
        (gcur_hbm, gprev_hbm, accin_hbm, dsq_hbm, tha_hbm, src_hbm, dst_hbm,
         gn_hbm, acc_hbm,
         src_v, dst_v, rows_v, rows_w, rows_x, rows_y, sv, gv, av,
         dsq_v,
         th_a, s_sp, sem_g0, sem_g1, sem_s0, sem_s1) = refs
    else:
        (gcur_hbm, gprev_hbm, accin_hbm, dsq_hbm, sd_hbm, tha_hbm, src_hbm,
         dst_hbm,
         acc_hbm,
         src_v, dst_v, rows_v, rows_w, rows_x, rows_y, sv, gv, av,
         dsq_v, aux_v,
         th_a, s_sp, sem_g0, sem_g1, sem_s0, sem_s1) = refs

    c = lax.axis_index("c")
    s = lax.axis_index("s")
    rbase = s * RPS

    lane16 = pl.ds(0, 16)
    if kind != "first_deg":
        pltpu.sync_copy(dsq_hbm.at[pl.ds(rbase, RPS)], dsq_v)
    if kind == "first":
        pltpu.sync_copy(d_hbm.at[pl.ds(rbase, RPS)], aux_v)
    elif kind == "final":
        pltpu.sync_copy(sd_hbm.at[pl.ds(rbase, RPS)], aux_v)
    pltpu.sync_copy(dst_hbm.at[s], dst_v)
    pltpu.sync_copy(tha_hbm, th_a)
    if kind in ("first", "first_deg", "pair"):
        pltpu.sync_copy(thb_hbm, th_b)
    _fill_zero(av, nv)
    zslice = av.at[pl.ds(0, ZH)]

    if kind == "first_deg":
        one = jnp.ones((16,), _f32)

        @pl.loop(0, B)
        def _(i):
            for v in range(nv):
                rows_v[i, pl.ds(16 * v, 16)] = one

        for t in range(NZ):
            pltpu.sync_copy(zslice, s_sp.at[pl.ds(rbase + t * ZH, ZH)])
        plsc.subcore_barrier()

        for kk in range(NBD // 16):
            pltpu.sync_copy(alli_hbm.at[s, pl.ds(kk * 16, 16)], alli_v)

            @pl.loop(0, 16)
            def _(j):
                pltpu.async_copy(rows_v, s_sp.at[alli_v.at[j]], sem_s0,
                                 add=True)

            @pl.loop(0, 16)
            def _(j):
                pltpu.make_async_copy(
                    rows_v, s_sp.at[alli_v.at[j]], sem_s0).wait()

        plsc.subcore_barrier()
        magic = jnp.full((16,), 0x5F3759DF, jnp.int32)
        half = jnp.full((16,), 0.5, _f32)
        threehalf = jnp.full((16,), 1.5, _f32)
        for t in range(NT):
            r0 = rbase + t * TS
            pltpu.sync_copy(s_sp.at[pl.ds(r0, TS)], sv)

            @pl.loop(0, TS)
            def _(i, t=t):
                x = jnp.maximum(sv[i, lane16], 1.0)
                y = plsc.bitcast(
                    magic - lax.shift_right_arithmetic(
                        plsc.bitcast(x, jnp.int32), 1), _f32)
                hx = half * x
                for _it in range(3):
                    y = y * (threehalf - hx * y * y)
                dsq_v[t * TS + i, lane16] = y * y
                aux_v[t * TS + i, lane16] = y
                st16[i, lane16] = x * y

            @pl.when(c == 0)
            def _(t=t, r0=r0):
                pltpu.sync_copy(st16, sd16_hbm.at[pl.ds(r0, TS)])

        @pl.when(c == 0)
        def _():
            pltpu.sync_copy(aux_v, d16_hbm.at[pl.ds(rbase, RPS)])
            pltpu.sync_copy(dsq_v, dsq16_hbm.at[pl.ds(rbase, RPS)])

    coef = -1.0 if kind in ("first", "first_deg") else -2.0
    substeps = 2 if kind == "pair" else 1
    for r in range(rounds):
      ch = c * rounds + r
      gb = ch * NPAD
      pltpu.sync_copy(src_hbm.at[ch, s], src_v)
      tva = [th_a[ch, pl.ds(16 * v, 16)] for v in range(nv)]
      first = kind in ("first", "first_deg")
      if first or kind == "pair":
          tvb = [th_b[ch, pl.ds(16 * v, 16)] for v in range(nv)]
      for sub in range(substeps):

        if first:
            for t in range(NT):
                r0 = rbase + t * TS
                pltpu.sync_copy(h_hbm.at[pl.ds(gb + r0, TS)], gv)

                @pl.loop(0, TS)
                def _(i, t=t):
                    dd = aux_v[t * TS + i, lane16]
                    for v in range(nv):
                        cs = pl.ds(16 * v, 16)
                        gv[i, cs] = dd * gv[i, cs]

                pltpu.sync_copy(gv, g0_hbm.at[pl.ds(gb + r0, TS)])

        if r > 0:
            _fill_zero(av, nv)
        for t in range(NZ):
            pltpu.sync_copy(zslice, s_sp.at[pl.ds(rbase + t * ZH, ZH)])
        plsc.subcore_barrier()

        if first:
            gsrc = g0_hbm
        elif kind == "pair":
            gsrc = gcur_hbm if sub == 0 else ga_hbm
        else:
            gsrc = gcur_hbm

        def _gather(j, buf, sem):
            return pltpu.async_copy(gsrc.at[src_v.at[j]], buf, sem)

        def _scatter(j, buf, sem):
            return pltpu.async_copy(buf, s_sp.at[dst_v.at[j]], sem, add=True)

        bufs = (rows_v, rows_w, rows_x, rows_y)
        sems = (sem_g0, sem_g1, sem_s0, sem_s1)

        def _wait_g(j, u):
            pltpu.make_async_copy(gsrc.at[src_v.at[j]], bufs[u],
                                  sems[u]).wait()

        def _wait_s(j, u):
            pltpu.make_async_copy(bufs[u], s_sp.at[dst_v.at[j]],
                                  sems[u]).wait()

        _gather(0, bufs[0], sems[0])
        _gather(1, bufs[1], sems[1])

        @pl.loop(0, NB // 4)
        def _(jj):
            j = 4 * jj
            for u in range(4):
                ju = j + u

                @pl.when(ju >= 2)
                def _(ju=ju, u=u):
                    _wait_s(ju - 2, (u + 2) % 4)

                @pl.when(ju + 2 < NB)
                def _(ju=ju, u=u):
                    _gather(ju + 2, bufs[(u + 2) % 4], sems[(u + 2) % 4])

                _wait_g(ju, u)
                _scatter(ju, bufs[u], sems[u])

        _wait_s(NB - 2, (NB - 2) % 4)
        _wait_s(NB - 1, (NB - 1) % 4)
        plsc.subcore_barrier()

        pair_a = kind == "pair" and sub == 0
        pair_b = kind == "pair" and sub == 1
        for t in range(NT):
            r0 = rbase + t * TS
            g0r = gb + r0
            pltpu.sync_copy(s_sp.at[pl.ds(r0, TS)], sv)
            if first:
                gp_src = g0_hbm
            elif pair_b:
                gp_src = gcur_hbm
            else:
                gp_src = gprev_hbm
            pltpu.sync_copy(gp_src.at[pl.ds(g0r, TS)], gv)
            if pair_b:
                pltpu.sync_copy(ga_hbm.at[pl.ds(g0r, TS)], rows_v)
            if not first and not pair_a:
                pltpu.sync_copy(accin_hbm.at[pl.ds(g0r, TS)], av)

            @pl.loop(0, TS)
            def _(i, t=t):
                m = coef * dsq_v[t * TS + i, lane16]
                if kind == "final":
                    sdd = aux_v[t * TS + i, lane16]
                for v in range(nv):
                    cs = pl.ds(16 * v, 16)
                    gn = m * sv[i, cs]
                    if first:
                        a = tva[v] * gv[i, cs] + tvb[v] * gn
                        av[i, cs] = a
                    elif pair_a:
                        gn = gn - gv[i, cs]
                    elif pair_b:
                        gn = gn - gv[i, cs]
                        a = (av[i, cs] + tva[v] * rows_v[i, cs]
                             + tvb[v] * gn)
                        av[i, cs] = a
                    else:
                        gn = gn - gv[i, cs]
                        a = av[i, cs] + tva[v] * gn
                        if kind == "final":
                            a = sdd * a
                        av[i, cs] = a
                    if kind != "final":
                        gv[i, cs] = gn

            if kind == "pair":
                out_g = ga_hbm if sub == 0 else gb_hbm
                pltpu.sync_copy(gv, out_g.at[pl.ds(g0r, TS)])
            elif kind != "final":
                pltpu.sync_copy(gv, gn_hbm.at[pl.ds(g0r, TS)])
            if not pair_a:
                pltpu.sync_copy(av, acc_hbm.at[pl.ds(g0r, TS)])
        plsc.subcore_barrier()


def _make_cheb_step(nch, chunk, kind):
    ts = 128 if kind in ("mid", "pair") else 64
    R = nch * NPAD
    n_out = {"first_deg": 3, "first": 3, "pair": 3, "mid": 2,
             "final": 1}[kind]
    outs = tuple(jax.ShapeDtypeStruct((R, chunk), _f32) for _ in range(n_out))
    if kind == "first_deg":
        outs = outs + tuple(
            jax.ShapeDtypeStruct((NPAD, 16), _f32) for _ in range(3))
    scratch = (
        pltpu.VMEM((NB, B), jnp.int32),
        pltpu.VMEM((NB, B), jnp.int32),
        pltpu.VMEM((B, chunk), _f32),
        pltpu.VMEM((B, chunk), _f32),
        pltpu.VMEM((B, chunk), _f32),
        pltpu.VMEM((B, chunk), _f32),
        pltpu.VMEM((ts, chunk), _f32),
        pltpu.VMEM((ts, chunk), _f32),
        pltpu.VMEM((ts, chunk), _f32),
        pltpu.VMEM((RPS, 16), _f32),
    )
    if kind in ("first", "first_deg", "final"):
        scratch = scratch + (pltpu.VMEM((RPS, 16), _f32),)
    scratch = scratch + (pltpu.VMEM((nch, chunk), _f32),)
    if kind in ("first", "first_deg", "pair"):
        scratch = scratch + (pltpu.VMEM((nch, chunk), _f32),)
    scratch = scratch + (
        pltpu.VMEM_SHARED((NPAD, chunk), _f32),
        pltpu.SemaphoreType.DMA,
        pltpu.SemaphoreType.DMA,
        pltpu.SemaphoreType.DMA,
        pltpu.SemaphoreType.DMA,
    )
    if kind == "first_deg":
        scratch = scratch + (
            pltpu.VMEM((16, B), jnp.int32),
            pltpu.VMEM((64, 16), _f32),
        )
    return pl.kernel(
        functools.partial(_cheb_body, nch, chunk, kind),
        out_type=outs,
        mesh=_mesh,
        scratch_types=scratch,
        compiler_params=pltpu.CompilerParams(
            use_tc_tiling_on_sc=False,
            needs_layout_passes=False,
        ),
    )


def _mm1_body(x_ref, w_ref, o_ref):
    o_ref[...] = jnp.dot(x_ref[...], w_ref[...],
                         preferred_element_type=_f32)


def _mm1(x_pad, W1):
    bm = 1024
    nb = NPAD // bm
    return pl.pallas_call(
        _mm1_body,
        out_shape=jax.ShapeDtypeStruct((4 * NPAD, 128), _f32),
        grid=(nb, 4),
        in_specs=[
            pl.BlockSpec((bm, F_IN), lambda i, ch: (i, 0)),
            pl.BlockSpec((F_IN, 128), lambda i, ch: (0, ch)),
        ],
        out_specs=pl.BlockSpec((bm, 128), lambda i, ch, nb=nb: (ch * nb + i, 0)),
    )(x_pad, W1)


def _mid_body(acc_ref, b_ref, w_ref, l1_ref, h2_ref):
    a = acc_ref[...] + b_ref[0:1, :]
    l1 = jnp.where(a > 0, a, jnp.exp(a) - 1.0)
    l1_ref[...] = l1

    @pl.when(pl.program_id(1) == 0)
    def _():
        h2_ref[...] = jnp.zeros_like(h2_ref)

    h2_ref[...] += jnp.dot(l1, w_ref[...], preferred_element_type=_f32)


def _mid(accT1, b1_2d, W2pad):
    bm = 1024
    nb = NPAD // bm
    return pl.pallas_call(
        _mid_body,
        out_shape=(
            jax.ShapeDtypeStruct((NPAD, HEADS * HIDDEN), _f32),
            jax.ShapeDtypeStruct((NPAD, 128), _f32),
        ),
        grid=(nb, 4),
        in_specs=[
            pl.BlockSpec((bm, 128), lambda i, ch, nb=nb: (ch * nb + i, 0)),
            pl.BlockSpec((8, 128), lambda i, ch: (ch, 0)),
            pl.BlockSpec((128, 128), lambda i, ch: (ch, 0)),
        ],
        out_specs=(
            pl.BlockSpec((bm, 128), lambda i, ch: (i, ch)),
            pl.BlockSpec((bm, 128), lambda i, ch: (i, 0)),
        ),
    )(accT1, b1_2d, W2pad)


def _final_body(x_ref, o_ref):
    x = x_ref[...]
    e = jnp.where(x > 0, x, jnp.exp(x) - 1.0)
    col = lax.broadcasted_iota(jnp.int32, x.shape, 1)
    valid = col < CLASSES
    em = jnp.where(valid, e, -1e30)
    m = jnp.max(em, axis=1, keepdims=True)
    z = jnp.where(valid, jnp.exp(em - m), 0.0)
    lse = jnp.log(jnp.sum(z, axis=1, keepdims=True))
    o_ref[...] = em - m - lse


def _final(l2pad):
    bm = 1024
    return pl.pallas_call(
        _final_body,
        out_shape=jax.ShapeDtypeStruct((NPAD, 128), _f32),
        grid=(NPAD // bm,),
        in_specs=[pl.BlockSpec((bm, 128), lambda i: (i, 0))],
        out_specs=pl.BlockSpec((bm, 128), lambda i: (i, 0)),
    )(l2pad)


_step1_l1 = _make_cheb_step(8, 64, "first_deg")
_pair_l1 = _make_cheb_step(8, 64, "pair")
_stepF_l1 = _make_cheb_step(8, 64, "final")
_step1_l2 = _make_cheb_step(2, 32, "first")
_pair_l2 = _make_cheb_step(2, 32, "pair")
_stepF_l2 = _make_cheb_step(2, 32, "final")


def _cheb_rest(g0, g1, acc, dsq16, sd16, th, src_off, dstp,
               pair_fn, final_fn):
    gprev, gcur = g0, g1
    for k in range(2, ORDER, 2):
        ga, gb, acc = pair_fn(gcur, gprev, acc, dsq16, th[k], th[k + 1],
                              src_off, dstp)
        gprev, gcur = ga, gb
    (accT,) = final_fn(gcur, gprev, acc, dsq16, sd16, th[ORDER],
                       src_off, dstp)
    return accT


def kernel(x, edge_index, W1, b1, theta1, W2, b2, theta2):
    src = edge_index[0]
    dst = edge_index[1]
    padw = NB * B - EPS

    srcp = jnp.pad(src.reshape(NSUB, EPS), ((0, 0), (0, padw)),
                   constant_values=JUNK).reshape(NSUB, NB, B)
    dstp = jnp.pad(dst.reshape(NSUB, EPS), ((0, 0), (0, padw)),
                   constant_values=JUNK).reshape(NSUB, NB, B)
    src1 = srcp[None] + (jnp.arange(8, dtype=jnp.int32) * NPAD)[:, None, None, None]
    src2 = srcp[None] + (jnp.arange(2, dtype=jnp.int32) * NPAD)[:, None, None, None]
    padd = NBD * B - 2 * EPS
    alli = jnp.pad(jnp.concatenate([src, dst]).reshape(NSUB, 2 * EPS),
                   ((0, 0), (0, padd)),
                   constant_values=JUNK).reshape(NSUB, NBD, B)

    th1 = jnp.repeat(theta1, HIDDEN, axis=0).T.reshape(ORDER + 1, 8, 64)
    th2 = jnp.broadcast_to(theta2.T, (ORDER + 1, 64)).reshape(ORDER + 1, 2, 32)

    x_pad = jnp.pad(x, ((0, NPAD - N), (0, 0)))
    h1c128 = _mm1(x_pad, W1)
    h1c = (h1c128.reshape(4, NPAD, 2, 64).transpose(0, 2, 1, 3)
           .reshape(8 * NPAD, 64))
    g0, g1, acc, d16, dsq16, sd16 = _step1_l1(h1c, th1[0], th1[1],
                                              src1, dstp, alli)
    accT1 = _cheb_rest(g0, g1, acc, dsq16, sd16, th1, src1, dstp,
                       _pair_l1, _stepF_l1)
    accT1c = (accT1.reshape(4, 2, NPAD, 64).transpose(0, 2, 1, 3)
              .reshape(4 * NPAD, 128))

    b1_2d = jnp.broadcast_to(b1.reshape(4, 1, 128), (4, 8, 128)).reshape(32, 128)
    W2pad = jnp.pad(W2.reshape(HEADS * HIDDEN, CLASSES),
                    ((0, 0), (0, 128 - CLASSES)))
    layer1_pad, h2 = _mid(accT1c, b1_2d, W2pad)
    layer1 = layer1_pad[:N]

    h2c = h2[:, :64].reshape(NPAD, 2, 32).transpose(1, 0, 2).reshape(2 * NPAD, 32)
    g0b, g1b, acc2 = _step1_l2(h2c, d16, dsq16, th2[0], th2[1], src2, dstp)
    accT2 = _cheb_rest(g0b, g1b, acc2, dsq16, sd16, th2, src2, dstp,
                       _pair_l2, _stepF_l2)
    accT2_std = accT2.reshape(2, NPAD, 32).transpose(1, 0, 2).reshape(NPAD, 64)
    layer2 = accT2_std[:N, :CLASSES] + b2

    l2pad = jnp.pad(accT2_std, ((0, 0), (0, 64)))
    l2pad = l2pad + jnp.pad(b2, (0, 88))[None, :]
    logp = _final(l2pad)[:N, :CLASSES]
    return (logp, layer2, layer1)
